# Initial kernel scaffold; baseline (speedup 1.0000x reference)
import jax
import jax.numpy as jnp
from jax.experimental import pallas as pl


def kernel(indices, eye):
    raise NotImplementedError("write your pallas kernel here")


# speedup vs baseline: 1.6893x; 1.6893x over previous
import functools

import jax
import jax.numpy as jnp
from jax import lax
from jax.experimental import pallas as pl
from jax.experimental.pallas import tpu as pltpu
from jax.experimental.pallas import tpu_sc as plsc

_N = 1000
_B = 4096 * 20
_NC = 2
_NS = 16
_NW = _NC * _NS
_RPW = _B // _NW
_R = 64
_NB = _RPW // _R
_L = 16


@functools.partial(
    pl.kernel,
    out_type=jax.ShapeDtypeStruct((_B * _N,), jnp.float32),
    mesh=plsc.VectorSubcoreMesh(core_axis_name="c", subcore_axis_name="s"),
    compiler_params=pltpu.CompilerParams(needs_layout_passes=False),
    scratch_types=[
        pltpu.VMEM((_R * _N,), jnp.float32),
        pltpu.VMEM((_R * _N,), jnp.float32),
        pltpu.VMEM((_R,), jnp.int32),
        pltpu.VMEM((_R,), jnp.int32),
        pltpu.SemaphoreType.DMA,
        pltpu.SemaphoreType.DMA,
    ],
)
def _one_hot_sc(idx_hbm, zeros_hbm, out_hbm, buf0, buf1, pos0, pos1, sem0, sem1):
    wid = lax.axis_index("s") * _NC + lax.axis_index("c")
    base = wid * _RPW

    bufs = [buf0, buf1]
    poss = [pos0, pos1]
    sems = [sem0, sem1]

    pltpu.sync_copy(zeros_hbm, buf0)
    pltpu.sync_copy(zeros_hbm, buf1)

    lane = lax.broadcasted_iota(jnp.int32, (_L,), 0)
    ones16 = jnp.ones((_L,), jnp.float32)
    zeros16 = jnp.zeros((_L,), jnp.float32)

    copies = [None, None]
    for t in range(_NB):
        slot = t % 2
        buf, pos = bufs[slot], poss[slot]
        if copies[slot] is not None:
            copies[slot].wait()
            for j in range(_R // _L):
                p = pos[pl.ds(j * _L, _L)]
                plsc.store_scatter(buf, [p], zeros16)
        row0 = base + t * _R
        pltpu.sync_copy(idx_hbm.at[pl.ds(row0, _R)], pos)
        for j in range(_R // _L):
            cls = pos[pl.ds(j * _L, _L)]
            p = (lane + j * _L) * _N + cls
            pos[pl.ds(j * _L, _L)] = p
            plsc.store_scatter(buf, [p], ones16)
        copies[slot] = pltpu.async_copy(
            buf, out_hbm.at[pl.ds(row0 * _N, _R * _N)], sems[slot]
        )
    copies[0].wait()
    copies[1].wait()


def kernel(indices, eye):
    del eye
    idx = indices.reshape(-1).astype(jnp.int32)
    zeros = jnp.zeros((_R * _N,), jnp.float32)
    out = _one_hot_sc(idx, zeros)
    return out.reshape(indices.shape + (_N,))


# speedup vs baseline: 1.6935x; 1.0025x over previous
import functools

import jax
import jax.numpy as jnp
from jax import lax
from jax.experimental import pallas as pl
from jax.experimental.pallas import tpu as pltpu
from jax.experimental.pallas import tpu_sc as plsc

_N = 1000
_B = 4096 * 20
_NC = 2
_NS = 16
_NW = _NC * _NS
_RPW = _B // _NW
_R = 64
_NB = _RPW // _R
_L = 16


@functools.partial(
    pl.kernel,
    out_type=jax.ShapeDtypeStruct((_B * _N,), jnp.float32),
    mesh=plsc.VectorSubcoreMesh(core_axis_name="c", subcore_axis_name="s"),
    compiler_params=pltpu.CompilerParams(needs_layout_passes=False),
    scratch_types=[
        pltpu.VMEM((_R * _N,), jnp.float32),
        pltpu.VMEM((_R * _N,), jnp.float32),
        pltpu.VMEM((_RPW,), jnp.int32),
        pltpu.SemaphoreType.DMA,
        pltpu.SemaphoreType.DMA,
    ],
)
def _one_hot_sc(idx_hbm, zeros_hbm, out_hbm, buf0, buf1, idxall, sem0, sem1):
    wid = lax.axis_index("s") * _NC + lax.axis_index("c")
    base = wid * _RPW

    bufs = [buf0, buf1]
    sems = [sem0, sem1]

    pltpu.sync_copy(idx_hbm.at[pl.ds(base, _RPW)], idxall)
    pltpu.sync_copy(zeros_hbm, buf0)
    pltpu.sync_copy(zeros_hbm, buf1)

    lane = lax.broadcasted_iota(jnp.int32, (_L,), 0)
    ones16 = jnp.ones((_L,), jnp.float32)
    zeros16 = jnp.zeros((_L,), jnp.float32)

    def positions(t, j):
        cls = idxall[pl.ds(t * _R + j * _L, _L)]
        return (lane + j * _L) * _N + cls

    copies = [None, None]
    for t in range(_NB):
        slot = t % 2
        buf = bufs[slot]
        if copies[slot] is not None:
            copies[slot].wait()
            for j in range(_R // _L):
                plsc.store_scatter(buf, [positions(t - 2, j)], zeros16)
        for j in range(_R // _L):
            plsc.store_scatter(buf, [positions(t, j)], ones16)
        row0 = base + t * _R
        copies[slot] = pltpu.async_copy(
            buf, out_hbm.at[pl.ds(row0 * _N, _R * _N)], sems[slot]
        )
    copies[0].wait()
    copies[1].wait()


def kernel(indices, eye):
    del eye
    idx = indices.reshape(-1).astype(jnp.int32)
    zeros = jnp.zeros((_R * _N,), jnp.float32)
    out = _one_hot_sc(idx, zeros)
    return out.reshape(indices.shape + (_N,))


# speedup vs baseline: 8.0429x; 4.7492x over previous
import functools

import jax
import jax.numpy as jnp
from jax import lax
from jax.experimental import pallas as pl
from jax.experimental.pallas import tpu as pltpu
from jax.experimental.pallas import tpu_sc as plsc

_N = 1000
_B = 4096
_T = 20
_NC = 2
_NS = 16
_NW = _NC * _NS
_BPW = _B // _NW
_L = 16
_C0 = 488
_C1 = _N - _C0


@functools.partial(
    pl.kernel,
    out_type=jax.ShapeDtypeStruct((_T, _N, _B), jnp.float32),
    mesh=plsc.VectorSubcoreMesh(core_axis_name="c", subcore_axis_name="s"),
    compiler_params=pltpu.CompilerParams(needs_layout_passes=False),
    scratch_types=[
        pltpu.VMEM((_C0, _BPW), jnp.float32),
        pltpu.VMEM((_C1, _BPW), jnp.float32),
        pltpu.VMEM((_BPW * _T,), jnp.int32),
        pltpu.SemaphoreType.DMA,
        pltpu.SemaphoreType.DMA,
    ],
)
def _one_hot_sc(idx_hbm, zeros_hbm, out_hbm, bufa, bufb, idxall, sema, semb):
    wid = lax.axis_index("s") * _NC + lax.axis_index("c")
    b0 = wid * _BPW

    pltpu.sync_copy(idx_hbm.at[pl.ds(b0 * _T, _BPW * _T)], idxall)
    pltpu.sync_copy(zeros_hbm.at[pl.ds(0, _C0), :], bufa)
    pltpu.sync_copy(zeros_hbm.at[pl.ds(0, _C1), :], bufb)

    lane = lax.broadcasted_iota(jnp.int32, (_L,), 0)
    ones16 = jnp.ones((_L,), jnp.float32)
    zeros16 = jnp.zeros((_L,), jnp.float32)

    def scatter_block(buf, j, c0, cw, val):
        for m in range(_BPW // _L):
            bl = lane + m * _L
            cls = plsc.load_gather(idxall, [bl * _T + j])
            mask = (cls >= c0) & (cls < c0 + cw)
            plsc.store_scatter(buf, [cls - c0, bl], val, mask=mask)

    chunks = ((bufa, 0, _C0, sema), (bufb, _C0, _C1, semb))
    copies = [None, None]
    for j in range(_T):
        for s, (buf, c0, cw, sem) in enumerate(chunks):
            if copies[s] is not None:
                copies[s].wait()
                scatter_block(buf, j - 1, c0, cw, zeros16)
            scatter_block(buf, j, c0, cw, ones16)
            copies[s] = pltpu.async_copy(
                buf, out_hbm.at[j, pl.ds(c0, cw), pl.ds(b0, _BPW)], sem
            )
    copies[0].wait()
    copies[1].wait()


def kernel(indices, eye):
    del eye
    idx = indices.reshape(-1).astype(jnp.int32)
    zeros = jnp.zeros((_C1, _BPW), jnp.float32)
    out = _one_hot_sc(idx, zeros)
    return jnp.transpose(out, (2, 0, 1))
